# baseline (speedup 1.0000x reference)
import functools

import jax
import jax.numpy as jnp
from jax import lax
from jax.experimental import pallas as pl
from jax.experimental.pallas import tpu as pltpu
from jax.experimental.pallas import tpu_sc as plsc

E = 8
EP = 128
NEG = -1e30

BT = 256
NBMAX = 2 * 2048 // BT + E

NC, NS, LANES = 2, 16, 16
NW = NC * NS


def _router_body(x_ref, gw_ref, bias_ref, iw_ref, ww_ref):
    x = x_ref[...]
    logits = jax.lax.dot_general(
        x, gw_ref[...], (((1,), (1,)), ((), ())),
        preferred_element_type=jnp.float32)
    rw = jax.nn.sigmoid(logits)
    scores = rw + bias_ref[...]
    idx = jax.lax.broadcasted_iota(jnp.int32, scores.shape, 1)
    m1 = jnp.max(scores, axis=1, keepdims=True)
    i1 = jnp.min(jnp.where(scores == m1, idx, EP), axis=1, keepdims=True)
    sel1 = idx == i1
    s2 = jnp.where(sel1, NEG, scores)
    m2 = jnp.max(s2, axis=1, keepdims=True)
    i2 = jnp.min(jnp.where(s2 == m2, idx, EP), axis=1, keepdims=True)
    sel2 = idx == i2
    w1 = jnp.sum(jnp.where(sel1, rw, 0.0), axis=1, keepdims=True)
    w2 = jnp.sum(jnp.where(sel2, rw, 0.0), axis=1, keepdims=True)
    den = w1 + w2 + 1e-6
    lane0 = idx == 0
    lane1 = idx == 1
    iw_ref[...] = (jnp.where(lane0, i1, 0) + jnp.where(lane1, i2, 0))
    ww_ref[...] = (jnp.where(lane0, w1 / den, 0.0)
                   + jnp.where(lane1, w2 / den, 0.0))


def _route_meta(iw, ww, T):
    e1, e2 = iw[:, 0], iw[:, 1]
    npad = NBMAX * BT
    ar = jnp.arange(E, dtype=jnp.int32)
    oh1 = (e1[:, None] == ar[None, :]).astype(jnp.int32)
    oh2 = (e2[:, None] == ar[None, :]).astype(jnp.int32)
    c1 = jnp.cumsum(oh1, axis=0)
    c2 = jnp.cumsum(oh2, axis=0)
    cnt1, cnt2 = c1[-1], c2[-1]
    rank1 = jnp.take_along_axis(c1, e1[:, None], 1)[:, 0] - 1
    rank2 = cnt1[e2] + jnp.take_along_axis(c2, e2[:, None], 1)[:, 0] - 1
    count = cnt1 + cnt2
    nb = (count + BT - 1) // BT
    bstart = jnp.concatenate(
        [jnp.zeros(1, jnp.int32), jnp.cumsum(nb)[:-1].astype(jnp.int32)])
    nbt = jnp.sum(nb).astype(jnp.int32)[None]
    pad_start = BT * bstart
    dst1 = pad_start[e1] + rank1
    dst2 = pad_start[e2] + rank2
    tok = jnp.arange(T, dtype=jnp.int32)
    tid = (jnp.zeros(npad, jnp.int32).at[dst1].set(tok).at[dst2].set(tok))
    wsl = (jnp.zeros(npad, jnp.float32).at[dst1].set(ww[:, 0])
           .at[dst2].set(ww[:, 1]))
    j = jnp.arange(NBMAX, dtype=jnp.int32)
    be = jnp.sum((j[:, None] >= bstart[None, :]).astype(jnp.int32), 1) - 1
    return (tid, wsl, be, nbt,
            dst1.astype(jnp.int32), dst2.astype(jnp.int32))


DCH = 16
DNB = 3


def _dispatch_body(tid_hbm, x_hbm, xs_hbm, idx_all, rows0, rows1, rows2,
                   semg0, semg1, semg2, sems0, sems1, sems2):
    wid = lax.axis_index("s") * NC + lax.axis_index("c")
    spw = tid_hbm.shape[0] // NW
    nch = spw // DCH
    base = wid * spw
    pltpu.sync_copy(tid_hbm.at[pl.ds(base, spw)], idx_all)
    rows = (rows0, rows1, rows2)
    semg = (semg0, semg1, semg2)
    sems = (sems0, sems1, sems2)

    def g(c):
        return pltpu.async_copy(
            x_hbm.at[idx_all.at[pl.ds(c * DCH, DCH)]], rows[c % DNB],
            semg[c % DNB])

    def s(c):
        return pltpu.async_copy(
            rows[c % DNB], xs_hbm.at[pl.ds(base + c * DCH, DCH)],
            sems[c % DNB])

    gd = [None] * nch
    sd = [None] * nch
    gd[0] = g(0)
    gd[1] = g(1)
    for c in range(nch):
        gd[c].wait()
        sd[c] = s(c)
        if c + 2 < nch:
            if c >= 1:
                sd[c - 1].wait()
            gd[c + 2] = g(c + 2)
    sd[nch - 3].wait()
    sd[nch - 2].wait()
    sd[nch - 1].wait()


CCH = 8
UNR = 4


def _combine_body(d1_hbm, d2_hbm, ys_hbm, out_hbm,
                  i1_all, i2_all, y1a, y1b, y2a, y2b,
                  sg1a, sg1b, sg2a, sg2b, ssa, ssb):
    wid = lax.axis_index("s") * NC + lax.axis_index("c")
    T = d1_hbm.shape[0]
    tpw = T // NW
    nch = tpw // CCH
    H = y1a.shape[1]
    base = wid * tpw
    pltpu.sync_copy(d1_hbm.at[pl.ds(base, tpw)], i1_all)
    pltpu.sync_copy(d2_hbm.at[pl.ds(base, tpw)], i2_all)
    y1 = (y1a, y1b)
    y2 = (y2a, y2b)
    sg1 = (sg1a, sg1b)
    sg2 = (sg2a, sg2b)
    ss = (ssa, ssb)

    def g(c):
        p = c % 2
        return (pltpu.async_copy(ys_hbm.at[i1_all.at[pl.ds(c * CCH, CCH)]],
                                 y1[p], sg1[p]),
                pltpu.async_copy(ys_hbm.at[i2_all.at[pl.ds(c * CCH, CCH)]],
                                 y2[p], sg2[p]))

    gd = [None] * nch
    sd = [None] * nch
    gd[0] = g(0)
    for c in range(nch):
        p = c % 2
        gd[c][0].wait()
        gd[c][1].wait()
        if c + 1 < nch:
            if c >= 1:
                sd[c - 1].wait()
            gd[c + 1] = g(c + 1)
        for r in range(CCH):

            def inner(jv, carry, r=r, p=p):
                for u in range(UNR):
                    sl = pl.ds((jv * UNR + u) * LANES, LANES)
                    y1[p][r, sl] = y1[p][r, sl] + y2[p][r, sl]
                return carry

            lax.fori_loop(0, H // (LANES * UNR), inner, 0)
        sd[c] = pltpu.async_copy(
            y1[p], out_hbm.at[pl.ds(base + c * CCH, CCH)], ss[p])
    sd[nch - 2].wait()
    sd[nch - 1].wait()


def _gffn_body(be_ref, nbt_ref, xs_ref, gp_ref, up_ref, dp_ref, ws_ref,
               ys_ref):
    i = pl.program_id(0)

    @pl.when(i < nbt_ref[0])
    def _():
        x = xs_ref[...]
        g = jax.lax.dot_general(x, gp_ref[0], (((1,), (1,)), ((), ())),
                                preferred_element_type=jnp.float32)
        u = jax.lax.dot_general(x, up_ref[0], (((1,), (1,)), ((), ())),
                                preferred_element_type=jnp.float32)
        h = g * jax.nn.sigmoid(g) * u
        y = jax.lax.dot_general(h, dp_ref[0], (((1,), (1,)), ((), ())),
                                preferred_element_type=jnp.float32)
        ys_ref[...] = y * ws_ref[...][:, 0:1]


@functools.partial(jax.jit, static_argnames=("interpret",))
def kernel(hidden_states, gate_w, gate_proj, up_proj, down_proj,
           expert_bias, interpret=False):
    B, S, H = hidden_states.shape
    T = B * S
    FF = gate_proj.shape[1]
    NPAD = NBMAX * BT
    x = hidden_states.reshape(T, H)

    gw_pad = jnp.zeros((EP, H), jnp.float32).at[:E].set(gate_w)
    bias_pad = jnp.full((1, EP), NEG, jnp.float32).at[0, :E].set(expert_bias)

    iw, ww = pl.pallas_call(
        _router_body,
        out_shape=(jax.ShapeDtypeStruct((T, EP), jnp.int32),
                   jax.ShapeDtypeStruct((T, EP), jnp.float32)),
        interpret=interpret,
    )(x, gw_pad, bias_pad)

    tid, wsl, be, nbt, dst1, dst2 = _route_meta(iw, ww, T)

    dispatch = pl.kernel(
        _dispatch_body,
        out_type=jax.ShapeDtypeStruct((NPAD, H), jnp.float32),
        mesh=plsc.VectorSubcoreMesh(core_axis_name="c", subcore_axis_name="s"),
        compiler_params=pltpu.CompilerParams(use_tc_tiling_on_sc=False),
        scratch_types=[
            pltpu.VMEM((NPAD // NW,), jnp.int32),
            pltpu.VMEM((DCH, H), jnp.float32),
            pltpu.VMEM((DCH, H), jnp.float32),
            pltpu.VMEM((DCH, H), jnp.float32),
            pltpu.SemaphoreType.DMA,
            pltpu.SemaphoreType.DMA,
            pltpu.SemaphoreType.DMA,
            pltpu.SemaphoreType.DMA,
            pltpu.SemaphoreType.DMA,
            pltpu.SemaphoreType.DMA,
        ],
    )
    xs = dispatch(tid, x)


    wsb = jnp.broadcast_to(wsl[:, None], (NPAD, EP))
    grid_spec = pltpu.PrefetchScalarGridSpec(
        num_scalar_prefetch=2,
        grid=(NBMAX,),
        in_specs=[
            pl.BlockSpec((BT, H), lambda i, be, nbt: (i, 0)),
            pl.BlockSpec((1, FF, H), lambda i, be, nbt: (be[i], 0, 0)),
            pl.BlockSpec((1, FF, H), lambda i, be, nbt: (be[i], 0, 0)),
            pl.BlockSpec((1, H, FF), lambda i, be, nbt: (be[i], 0, 0)),
            pl.BlockSpec((BT, EP), lambda i, be, nbt: (i, 0)),
        ],
        out_specs=pl.BlockSpec((BT, H), lambda i, be, nbt: (i, 0)),
    )
    ys = pl.pallas_call(
        _gffn_body,
        grid_spec=grid_spec,
        out_shape=jax.ShapeDtypeStruct((NPAD, H), jnp.float32),
        compiler_params=pltpu.CompilerParams(
            dimension_semantics=("arbitrary",)),
        interpret=interpret,
    )(be, nbt, xs, gate_proj, up_proj, down_proj, wsb)

    combine = pl.kernel(
        _combine_body,
        out_type=jax.ShapeDtypeStruct((T, H), jnp.float32),
        mesh=plsc.VectorSubcoreMesh(core_axis_name="c", subcore_axis_name="s"),
        scratch_types=[
            pltpu.VMEM((T // NW,), jnp.int32),
            pltpu.VMEM((T // NW,), jnp.int32),
            pltpu.VMEM((CCH, H), jnp.float32),
            pltpu.VMEM((CCH, H), jnp.float32),
            pltpu.VMEM((CCH, H), jnp.float32),
            pltpu.VMEM((CCH, H), jnp.float32),
            pltpu.SemaphoreType.DMA,
            pltpu.SemaphoreType.DMA,
            pltpu.SemaphoreType.DMA,
            pltpu.SemaphoreType.DMA,
            pltpu.SemaphoreType.DMA,
            pltpu.SemaphoreType.DMA,
        ],
    )
    out = combine(dst1, dst2, ys)

    return out.reshape(B, S, H)


# speedup vs baseline: 1.3538x; 1.3538x over previous
import functools

import jax
import jax.numpy as jnp
from jax import lax
from jax.experimental import pallas as pl
from jax.experimental.pallas import tpu as pltpu
from jax.experimental.pallas import tpu_sc as plsc

E = 8
EP = 128
NEG = -1e30

BT = 256
NBMAX = 2 * 2048 // BT + E

NC, NS, LANES = 2, 16, 16
NW = NC * NS


def _router_body(x_ref, gw_ref, bias_ref, iw_ref, ww_ref):
    x = x_ref[...]
    logits = jax.lax.dot_general(
        x, gw_ref[...], (((1,), (1,)), ((), ())),
        preferred_element_type=jnp.float32)
    rw = jax.nn.sigmoid(logits)
    scores = rw + bias_ref[...]
    idx = jax.lax.broadcasted_iota(jnp.int32, scores.shape, 1)
    m1 = jnp.max(scores, axis=1, keepdims=True)
    i1 = jnp.min(jnp.where(scores == m1, idx, EP), axis=1, keepdims=True)
    sel1 = idx == i1
    s2 = jnp.where(sel1, NEG, scores)
    m2 = jnp.max(s2, axis=1, keepdims=True)
    i2 = jnp.min(jnp.where(s2 == m2, idx, EP), axis=1, keepdims=True)
    sel2 = idx == i2
    w1 = jnp.sum(jnp.where(sel1, rw, 0.0), axis=1, keepdims=True)
    w2 = jnp.sum(jnp.where(sel2, rw, 0.0), axis=1, keepdims=True)
    den = w1 + w2 + 1e-6
    lane0 = idx == 0
    lane1 = idx == 1
    iw_ref[...] = (jnp.where(lane0, i1, 0) + jnp.where(lane1, i2, 0))
    ww_ref[...] = (jnp.where(lane0, w1 / den, 0.0)
                   + jnp.where(lane1, w2 / den, 0.0))


def _route_meta(iw, ww, T):
    e1, e2 = iw[:, 0], iw[:, 1]
    npad = NBMAX * BT
    ar = jnp.arange(E, dtype=jnp.int32)
    oh1 = (e1[:, None] == ar[None, :]).astype(jnp.int32)
    oh2 = (e2[:, None] == ar[None, :]).astype(jnp.int32)
    c1 = jnp.cumsum(oh1, axis=0)
    c2 = jnp.cumsum(oh2, axis=0)
    cnt1, cnt2 = c1[-1], c2[-1]
    rank1 = jnp.take_along_axis(c1, e1[:, None], 1)[:, 0] - 1
    rank2 = cnt1[e2] + jnp.take_along_axis(c2, e2[:, None], 1)[:, 0] - 1
    count = cnt1 + cnt2
    nb = (count + BT - 1) // BT
    bstart = jnp.concatenate(
        [jnp.zeros(1, jnp.int32), jnp.cumsum(nb)[:-1].astype(jnp.int32)])
    nbt = jnp.sum(nb).astype(jnp.int32)[None]
    pad_start = BT * bstart
    dst1 = pad_start[e1] + rank1
    dst2 = pad_start[e2] + rank2
    tok = jnp.arange(T, dtype=jnp.int32)
    tid = (jnp.zeros(npad, jnp.int32).at[dst1].set(tok).at[dst2].set(tok))
    wsl = (jnp.zeros(npad, jnp.float32).at[dst1].set(ww[:, 0])
           .at[dst2].set(ww[:, 1]))
    j = jnp.arange(NBMAX, dtype=jnp.int32)
    be = jnp.sum((j[:, None] >= bstart[None, :]).astype(jnp.int32), 1) - 1
    return (tid, wsl, be, nbt,
            dst1.astype(jnp.int32), dst2.astype(jnp.int32))


DCH = 16
DNB = 3


def _dispatch_body(tid_hbm, x_hbm, xs_hbm, idx_all, rows0, rows1, rows2,
                   semg0, semg1, semg2, sems0, sems1, sems2):
    wid = lax.axis_index("s") * NC + lax.axis_index("c")
    spw = tid_hbm.shape[0] // NW
    nch = spw // DCH
    base = wid * spw
    pltpu.sync_copy(tid_hbm.at[pl.ds(base, spw)], idx_all)
    rows = (rows0, rows1, rows2)
    semg = (semg0, semg1, semg2)
    sems = (sems0, sems1, sems2)

    def g(c):
        return pltpu.async_copy(
            x_hbm.at[idx_all.at[pl.ds(c * DCH, DCH)]], rows[c % DNB],
            semg[c % DNB])

    def s(c):
        return pltpu.async_copy(
            rows[c % DNB], xs_hbm.at[pl.ds(base + c * DCH, DCH)],
            sems[c % DNB])

    gd = [None] * nch
    sd = [None] * nch
    gd[0] = g(0)
    gd[1] = g(1)
    for c in range(nch):
        gd[c].wait()
        sd[c] = s(c)
        if c + 2 < nch:
            if c >= 1:
                sd[c - 1].wait()
            gd[c + 2] = g(c + 2)
    sd[nch - 3].wait()
    sd[nch - 2].wait()
    sd[nch - 1].wait()


CCH = 8
UNR = 4


def _combine_body(d1_hbm, d2_hbm, ys_hbm, out_hbm,
                  i1_all, i2_all, y1a, y1b, y2a, y2b,
                  sg1a, sg1b, sg2a, sg2b, ssa, ssb):
    wid = lax.axis_index("s") * NC + lax.axis_index("c")
    T = d1_hbm.shape[0]
    tpw = T // NW
    nch = tpw // CCH
    H = y1a.shape[1]
    base = wid * tpw
    pltpu.sync_copy(d1_hbm.at[pl.ds(base, tpw)], i1_all)
    pltpu.sync_copy(d2_hbm.at[pl.ds(base, tpw)], i2_all)
    y1 = (y1a, y1b)
    y2 = (y2a, y2b)
    sg1 = (sg1a, sg1b)
    sg2 = (sg2a, sg2b)
    ss = (ssa, ssb)

    def g(c):
        p = c % 2
        return (pltpu.async_copy(ys_hbm.at[i1_all.at[pl.ds(c * CCH, CCH)]],
                                 y1[p], sg1[p]),
                pltpu.async_copy(ys_hbm.at[i2_all.at[pl.ds(c * CCH, CCH)]],
                                 y2[p], sg2[p]))

    gd = [None] * nch
    sd = [None] * nch
    gd[0] = g(0)
    for c in range(nch):
        p = c % 2
        gd[c][0].wait()
        gd[c][1].wait()
        if c + 1 < nch:
            if c >= 1:
                sd[c - 1].wait()
            gd[c + 1] = g(c + 1)
        for r in range(CCH):

            def inner(jv, carry, r=r, p=p):
                for u in range(UNR):
                    sl = pl.ds((jv * UNR + u) * LANES, LANES)
                    y1[p][r, sl] = y1[p][r, sl] + y2[p][r, sl]
                return carry

            lax.fori_loop(0, H // (LANES * UNR), inner, 0)
        sd[c] = pltpu.async_copy(
            y1[p], out_hbm.at[pl.ds(base + c * CCH, CCH)], ss[p])
    sd[nch - 2].wait()
    sd[nch - 1].wait()


def _gffn_body(be_ref, nbt_ref, tid_ref, gp_ref, up_ref, dp_ref, ws_ref,
               x_hbm, ys_ref, xbuf, sem0, sem1):
    i = pl.program_id(0)
    BTc, H = ys_ref.shape

    def issue(blk, par, sem):
        def one(r, carry):
            t = tid_ref[blk * BTc + r]
            pltpu.make_async_copy(
                x_hbm.at[pl.ds(t, 1)],
                xbuf.at[pl.ds(par * BTc + r, 1)], sem).start()
            return carry
        lax.fori_loop(0, BTc, one, 0)

    def drain(sem):
        def one(r, carry):
            pltpu.make_async_copy(
                x_hbm.at[pl.ds(0, 1)], xbuf.at[pl.ds(0, 1)], sem).wait()
            return carry
        lax.fori_loop(0, BTc, one, 0)

    nbt = nbt_ref[0]

    @pl.when(i == 0)
    def _():
        issue(0, 0, sem0)

    @pl.when((i + 1 < nbt) & (i % 2 == 0))
    def _():
        issue(i + 1, 1, sem1)

    @pl.when((i + 1 < nbt) & (i % 2 == 1))
    def _():
        issue(i + 1, 0, sem0)

    @pl.when(i < nbt)
    def _():

        @pl.when(i % 2 == 0)
        def _():
            drain(sem0)

        @pl.when(i % 2 == 1)
        def _():
            drain(sem1)

        x = xbuf[pl.ds((i % 2) * BTc, BTc), :]
        g = jax.lax.dot_general(x, gp_ref[0], (((1,), (1,)), ((), ())),
                                preferred_element_type=jnp.float32)
        u = jax.lax.dot_general(x, up_ref[0], (((1,), (1,)), ((), ())),
                                preferred_element_type=jnp.float32)
        h = g * jax.nn.sigmoid(g) * u
        y = jax.lax.dot_general(h, dp_ref[0], (((1,), (1,)), ((), ())),
                                preferred_element_type=jnp.float32)
        ys_ref[...] = y * ws_ref[...][:, 0:1]


@functools.partial(jax.jit, static_argnames=("interpret",))
def kernel(hidden_states, gate_w, gate_proj, up_proj, down_proj,
           expert_bias, interpret=False):
    B, S, H = hidden_states.shape
    T = B * S
    FF = gate_proj.shape[1]
    NPAD = NBMAX * BT
    x = hidden_states.reshape(T, H)

    gw_pad = jnp.zeros((EP, H), jnp.float32).at[:E].set(gate_w)
    bias_pad = jnp.full((1, EP), NEG, jnp.float32).at[0, :E].set(expert_bias)

    iw, ww = pl.pallas_call(
        _router_body,
        out_shape=(jax.ShapeDtypeStruct((T, EP), jnp.int32),
                   jax.ShapeDtypeStruct((T, EP), jnp.float32)),
        interpret=interpret,
    )(x, gw_pad, bias_pad)

    tid, wsl, be, nbt, dst1, dst2 = _route_meta(iw, ww, T)

    wsb = jnp.broadcast_to(wsl[:, None], (NPAD, EP))
    grid_spec = pltpu.PrefetchScalarGridSpec(
        num_scalar_prefetch=3,
        grid=(NBMAX,),
        in_specs=[
            pl.BlockSpec((1, FF, H), lambda i, be, nbt, tid: (be[i], 0, 0)),
            pl.BlockSpec((1, FF, H), lambda i, be, nbt, tid: (be[i], 0, 0)),
            pl.BlockSpec((1, H, FF), lambda i, be, nbt, tid: (be[i], 0, 0)),
            pl.BlockSpec((BT, EP), lambda i, be, nbt, tid: (i, 0)),
            pl.BlockSpec(memory_space=pl.ANY),
        ],
        out_specs=pl.BlockSpec((BT, H), lambda i, be, nbt, tid: (i, 0)),
        scratch_shapes=[
            pltpu.VMEM((2 * BT, H), jnp.float32),
            pltpu.SemaphoreType.DMA,
            pltpu.SemaphoreType.DMA,
        ],
    )
    ys = pl.pallas_call(
        _gffn_body,
        grid_spec=grid_spec,
        out_shape=jax.ShapeDtypeStruct((NPAD, H), jnp.float32),
        compiler_params=pltpu.CompilerParams(
            dimension_semantics=("arbitrary",)),
        interpret=interpret,
    )(be, nbt, tid, gate_proj, up_proj, down_proj, wsb, x)

    combine = pl.kernel(
        _combine_body,
        out_type=jax.ShapeDtypeStruct((T, H), jnp.float32),
        mesh=plsc.VectorSubcoreMesh(core_axis_name="c", subcore_axis_name="s"),
        scratch_types=[
            pltpu.VMEM((T // NW,), jnp.int32),
            pltpu.VMEM((T // NW,), jnp.int32),
            pltpu.VMEM((CCH, H), jnp.float32),
            pltpu.VMEM((CCH, H), jnp.float32),
            pltpu.VMEM((CCH, H), jnp.float32),
            pltpu.VMEM((CCH, H), jnp.float32),
            pltpu.SemaphoreType.DMA,
            pltpu.SemaphoreType.DMA,
            pltpu.SemaphoreType.DMA,
            pltpu.SemaphoreType.DMA,
            pltpu.SemaphoreType.DMA,
            pltpu.SemaphoreType.DMA,
        ],
    )
    out = combine(dst1, dst2, ys)

    return out.reshape(B, S, H)


# speedup vs baseline: 1.4240x; 1.0519x over previous
import functools

import jax
import jax.numpy as jnp
from jax import lax
from jax.experimental import pallas as pl
from jax.experimental.pallas import tpu as pltpu
from jax.experimental.pallas import tpu_sc as plsc

E = 8
EP = 128
NEG = -1e30

BT = 256
NBMAX = 2 * 2048 // BT + E

NC, NS, LANES = 2, 16, 16
NW = NC * NS


def _router_body(x_ref, gw_ref, bias_ref, iw_ref, ww_ref):
    x = x_ref[...]
    logits = jax.lax.dot_general(
        x, gw_ref[...], (((1,), (1,)), ((), ())),
        preferred_element_type=jnp.float32)
    rw = jax.nn.sigmoid(logits)
    scores = rw + bias_ref[...]
    idx = jax.lax.broadcasted_iota(jnp.int32, scores.shape, 1)
    m1 = jnp.max(scores, axis=1, keepdims=True)
    i1 = jnp.min(jnp.where(scores == m1, idx, EP), axis=1, keepdims=True)
    sel1 = idx == i1
    s2 = jnp.where(sel1, NEG, scores)
    m2 = jnp.max(s2, axis=1, keepdims=True)
    i2 = jnp.min(jnp.where(s2 == m2, idx, EP), axis=1, keepdims=True)
    sel2 = idx == i2
    w1 = jnp.sum(jnp.where(sel1, rw, 0.0), axis=1, keepdims=True)
    w2 = jnp.sum(jnp.where(sel2, rw, 0.0), axis=1, keepdims=True)
    den = w1 + w2 + 1e-6
    lane0 = idx == 0
    lane1 = idx == 1
    iw_ref[...] = (jnp.where(lane0, i1, 0) + jnp.where(lane1, i2, 0))
    ww_ref[...] = (jnp.where(lane0, w1 / den, 0.0)
                   + jnp.where(lane1, w2 / den, 0.0))


def _route_meta(iw, ww, T):
    e1, e2 = iw[:, 0], iw[:, 1]
    npad = NBMAX * BT
    ar = jnp.arange(E, dtype=jnp.int32)
    oh1 = (e1[:, None] == ar[None, :]).astype(jnp.int32)
    oh2 = (e2[:, None] == ar[None, :]).astype(jnp.int32)
    c1 = jnp.cumsum(oh1, axis=0)
    c2 = jnp.cumsum(oh2, axis=0)
    cnt1, cnt2 = c1[-1], c2[-1]
    rank1 = jnp.take_along_axis(c1, e1[:, None], 1)[:, 0] - 1
    rank2 = cnt1[e2] + jnp.take_along_axis(c2, e2[:, None], 1)[:, 0] - 1
    count = cnt1 + cnt2
    nb = (count + BT - 1) // BT
    bstart = jnp.concatenate(
        [jnp.zeros(1, jnp.int32), jnp.cumsum(nb)[:-1].astype(jnp.int32)])
    nbt = jnp.sum(nb).astype(jnp.int32)[None]
    pad_start = BT * bstart
    dst1 = pad_start[e1] + rank1
    dst2 = pad_start[e2] + rank2
    tok = jnp.arange(T, dtype=jnp.int32)
    tid = (jnp.zeros(npad, jnp.int32).at[dst1].set(tok).at[dst2].set(tok))
    wsl = (jnp.zeros(npad, jnp.float32).at[dst1].set(ww[:, 0])
           .at[dst2].set(ww[:, 1]))
    j = jnp.arange(NBMAX, dtype=jnp.int32)
    be = jnp.sum((j[:, None] >= bstart[None, :]).astype(jnp.int32), 1) - 1
    return (tid, wsl, be, nbt,
            dst1.astype(jnp.int32), dst2.astype(jnp.int32))


DCH = 16
DNB = 3


def _dispatch_body(tid_hbm, x_hbm, xs_hbm, idx_all, rows0, rows1, rows2,
                   semg0, semg1, semg2, sems0, sems1, sems2):
    wid = lax.axis_index("s") * NC + lax.axis_index("c")
    spw = tid_hbm.shape[0] // NW
    nch = spw // DCH
    base = wid * spw
    pltpu.sync_copy(tid_hbm.at[pl.ds(base, spw)], idx_all)
    rows = (rows0, rows1, rows2)
    semg = (semg0, semg1, semg2)
    sems = (sems0, sems1, sems2)

    def g(c):
        return pltpu.async_copy(
            x_hbm.at[idx_all.at[pl.ds(c * DCH, DCH)]], rows[c % DNB],
            semg[c % DNB])

    def s(c):
        return pltpu.async_copy(
            rows[c % DNB], xs_hbm.at[pl.ds(base + c * DCH, DCH)],
            sems[c % DNB])

    gd = [None] * nch
    sd = [None] * nch
    gd[0] = g(0)
    gd[1] = g(1)
    for c in range(nch):
        gd[c].wait()
        sd[c] = s(c)
        if c + 2 < nch:
            if c >= 1:
                sd[c - 1].wait()
            gd[c + 2] = g(c + 2)
    sd[nch - 3].wait()
    sd[nch - 2].wait()
    sd[nch - 1].wait()


CCH = 8
UNR = 4


def _combine_body(d1_hbm, d2_hbm, ys_hbm, out_hbm,
                  i1_all, i2_all, y1a, y1b, y2a, y2b,
                  sg1a, sg1b, sg2a, sg2b, ssa, ssb):
    wid = lax.axis_index("s") * NC + lax.axis_index("c")
    T = d1_hbm.shape[0]
    tpw = T // NW
    nch = tpw // CCH
    H = y1a.shape[1]
    base = wid * tpw
    pltpu.sync_copy(d1_hbm.at[pl.ds(base, tpw)], i1_all)
    pltpu.sync_copy(d2_hbm.at[pl.ds(base, tpw)], i2_all)
    y1 = (y1a, y1b)
    y2 = (y2a, y2b)
    sg1 = (sg1a, sg1b)
    sg2 = (sg2a, sg2b)
    ss = (ssa, ssb)

    def g(c):
        p = c % 2
        return (pltpu.async_copy(ys_hbm.at[i1_all.at[pl.ds(c * CCH, CCH)]],
                                 y1[p], sg1[p]),
                pltpu.async_copy(ys_hbm.at[i2_all.at[pl.ds(c * CCH, CCH)]],
                                 y2[p], sg2[p]))

    gd = [None] * nch
    sd = [None] * nch
    gd[0] = g(0)
    for c in range(nch):
        p = c % 2
        gd[c][0].wait()
        gd[c][1].wait()
        if c + 1 < nch:
            if c >= 1:
                sd[c - 1].wait()
            gd[c + 1] = g(c + 1)
        for r in range(CCH):

            def inner(jv, carry, r=r, p=p):
                for u in range(UNR):
                    sl = pl.ds((jv * UNR + u) * LANES, LANES)
                    y1[p][r, sl] = y1[p][r, sl] + y2[p][r, sl]
                return carry

            lax.fori_loop(0, H // (LANES * UNR), inner, 0)
        sd[c] = pltpu.async_copy(
            y1[p], out_hbm.at[pl.ds(base + c * CCH, CCH)], ss[p])
    sd[nch - 2].wait()
    sd[nch - 1].wait()


def _gffn_body(be_ref, nbt_ref, tid_ref, gp_ref, up_ref, dp_ref, ws_ref,
               x_hbm, ys_ref, xbuf, sem0, sem1):
    i = pl.program_id(0)
    BTc, H = ys_ref.shape

    def issue(blk, par, sem):
        def one(r8, carry):
            for u in range(8):
                r = r8 * 8 + u
                t = tid_ref[blk * BTc + r]
                pltpu.make_async_copy(
                    x_hbm.at[pl.ds(t, 1)],
                    xbuf.at[pl.ds(par * BTc + r, 1)], sem).start()
            return carry
        lax.fori_loop(0, BTc // 8, one, 0)

    def drain(par, sem):
        pltpu.make_async_copy(
            x_hbm.at[pl.ds(0, BTc)],
            xbuf.at[pl.ds(par * BTc, BTc)], sem).wait()

    nbt = nbt_ref[0]

    @pl.when(i == 0)
    def _():
        issue(0, 0, sem0)

    @pl.when((i + 1 < nbt) & (i % 2 == 0))
    def _():
        issue(i + 1, 1, sem1)

    @pl.when((i + 1 < nbt) & (i % 2 == 1))
    def _():
        issue(i + 1, 0, sem0)

    @pl.when(i < nbt)
    def _():

        @pl.when(i % 2 == 0)
        def _():
            drain(0, sem0)

        @pl.when(i % 2 == 1)
        def _():
            drain(1, sem1)

        x = xbuf[pl.ds((i % 2) * BTc, BTc), :]
        g = jax.lax.dot_general(x, gp_ref[0], (((1,), (1,)), ((), ())),
                                preferred_element_type=jnp.float32)
        u = jax.lax.dot_general(x, up_ref[0], (((1,), (1,)), ((), ())),
                                preferred_element_type=jnp.float32)
        h = g * jax.nn.sigmoid(g) * u
        y = jax.lax.dot_general(h, dp_ref[0], (((1,), (1,)), ((), ())),
                                preferred_element_type=jnp.float32)
        ys_ref[...] = y * ws_ref[...][:, 0:1]


@functools.partial(jax.jit, static_argnames=("interpret",))
def kernel(hidden_states, gate_w, gate_proj, up_proj, down_proj,
           expert_bias, interpret=False):
    B, S, H = hidden_states.shape
    T = B * S
    FF = gate_proj.shape[1]
    NPAD = NBMAX * BT
    x = hidden_states.reshape(T, H)

    gw_pad = jnp.zeros((EP, H), jnp.float32).at[:E].set(gate_w)
    bias_pad = jnp.full((1, EP), NEG, jnp.float32).at[0, :E].set(expert_bias)

    iw, ww = pl.pallas_call(
        _router_body,
        out_shape=(jax.ShapeDtypeStruct((T, EP), jnp.int32),
                   jax.ShapeDtypeStruct((T, EP), jnp.float32)),
        interpret=interpret,
    )(x, gw_pad, bias_pad)

    tid, wsl, be, nbt, dst1, dst2 = _route_meta(iw, ww, T)

    wsb = jnp.broadcast_to(wsl[:, None], (NPAD, EP))
    grid_spec = pltpu.PrefetchScalarGridSpec(
        num_scalar_prefetch=3,
        grid=(NBMAX,),
        in_specs=[
            pl.BlockSpec((1, FF, H), lambda i, be, nbt, tid: (be[i], 0, 0)),
            pl.BlockSpec((1, FF, H), lambda i, be, nbt, tid: (be[i], 0, 0)),
            pl.BlockSpec((1, H, FF), lambda i, be, nbt, tid: (be[i], 0, 0)),
            pl.BlockSpec((BT, EP), lambda i, be, nbt, tid: (i, 0)),
            pl.BlockSpec(memory_space=pl.ANY),
        ],
        out_specs=pl.BlockSpec((BT, H), lambda i, be, nbt, tid: (i, 0)),
        scratch_shapes=[
            pltpu.VMEM((2 * BT, H), jnp.float32),
            pltpu.SemaphoreType.DMA,
            pltpu.SemaphoreType.DMA,
        ],
    )
    ys = pl.pallas_call(
        _gffn_body,
        grid_spec=grid_spec,
        out_shape=jax.ShapeDtypeStruct((NPAD, H), jnp.float32),
        compiler_params=pltpu.CompilerParams(
            dimension_semantics=("arbitrary",)),
        interpret=interpret,
    )(be, nbt, tid, gate_proj, up_proj, down_proj, wsb, x)

    combine = pl.kernel(
        _combine_body,
        out_type=jax.ShapeDtypeStruct((T, H), jnp.float32),
        mesh=plsc.VectorSubcoreMesh(core_axis_name="c", subcore_axis_name="s"),
        scratch_types=[
            pltpu.VMEM((T // NW,), jnp.int32),
            pltpu.VMEM((T // NW,), jnp.int32),
            pltpu.VMEM((CCH, H), jnp.float32),
            pltpu.VMEM((CCH, H), jnp.float32),
            pltpu.VMEM((CCH, H), jnp.float32),
            pltpu.VMEM((CCH, H), jnp.float32),
            pltpu.SemaphoreType.DMA,
            pltpu.SemaphoreType.DMA,
            pltpu.SemaphoreType.DMA,
            pltpu.SemaphoreType.DMA,
            pltpu.SemaphoreType.DMA,
            pltpu.SemaphoreType.DMA,
        ],
    )
    out = combine(dst1, dst2, ys)

    return out.reshape(B, S, H)


# speedup vs baseline: 1.4315x; 1.0052x over previous
import functools

import jax
import jax.numpy as jnp
from jax import lax
from jax.experimental import pallas as pl
from jax.experimental.pallas import tpu as pltpu
from jax.experimental.pallas import tpu_sc as plsc

E = 8
EP = 128
NEG = -1e30

BT = 256
NBMAX = 2 * 2048 // BT + E

NC, NS, LANES = 2, 16, 16
NW = NC * NS


def _router_body(x_ref, gw_ref, bias_ref, iw_ref, ww_ref):
    x = x_ref[...]
    logits = jax.lax.dot_general(
        x, gw_ref[...], (((1,), (1,)), ((), ())),
        preferred_element_type=jnp.float32)
    rw = jax.nn.sigmoid(logits)
    scores = rw + bias_ref[...]
    idx = jax.lax.broadcasted_iota(jnp.int32, scores.shape, 1)
    m1 = jnp.max(scores, axis=1, keepdims=True)
    i1 = jnp.min(jnp.where(scores == m1, idx, EP), axis=1, keepdims=True)
    sel1 = idx == i1
    s2 = jnp.where(sel1, NEG, scores)
    m2 = jnp.max(s2, axis=1, keepdims=True)
    i2 = jnp.min(jnp.where(s2 == m2, idx, EP), axis=1, keepdims=True)
    sel2 = idx == i2
    w1 = jnp.sum(jnp.where(sel1, rw, 0.0), axis=1, keepdims=True)
    w2 = jnp.sum(jnp.where(sel2, rw, 0.0), axis=1, keepdims=True)
    den = w1 + w2 + 1e-6
    lane0 = idx == 0
    lane1 = idx == 1
    iw_ref[...] = (jnp.where(lane0, i1, 0) + jnp.where(lane1, i2, 0))
    ww_ref[...] = (jnp.where(lane0, w1 / den, 0.0)
                   + jnp.where(lane1, w2 / den, 0.0))


def _route_meta(iw, ww, T):
    e1, e2 = iw[:, 0], iw[:, 1]
    npad = NBMAX * BT
    ar = jnp.arange(E, dtype=jnp.int32)
    oh1 = (e1[:, None] == ar[None, :]).astype(jnp.int32)
    oh2 = (e2[:, None] == ar[None, :]).astype(jnp.int32)
    c1 = jnp.cumsum(oh1, axis=0)
    c2 = jnp.cumsum(oh2, axis=0)
    cnt1, cnt2 = c1[-1], c2[-1]
    rank1 = jnp.take_along_axis(c1, e1[:, None], 1)[:, 0] - 1
    rank2 = cnt1[e2] + jnp.take_along_axis(c2, e2[:, None], 1)[:, 0] - 1
    count = cnt1 + cnt2
    nb = (count + BT - 1) // BT
    bstart = jnp.concatenate(
        [jnp.zeros(1, jnp.int32), jnp.cumsum(nb)[:-1].astype(jnp.int32)])
    nbt = jnp.sum(nb).astype(jnp.int32)[None]
    pad_start = BT * bstart
    dst1 = pad_start[e1] + rank1
    dst2 = pad_start[e2] + rank2
    tok = jnp.arange(T, dtype=jnp.int32)
    tid = (jnp.zeros(npad, jnp.int32).at[dst1].set(tok).at[dst2].set(tok))
    wsl = (jnp.zeros(npad, jnp.float32).at[dst1].set(ww[:, 0])
           .at[dst2].set(ww[:, 1]))
    j = jnp.arange(NBMAX, dtype=jnp.int32)
    be = jnp.sum((j[:, None] >= bstart[None, :]).astype(jnp.int32), 1) - 1
    return (tid, wsl, be, nbt,
            dst1.astype(jnp.int32), dst2.astype(jnp.int32))


CCH = 8
UNR = 4


def _combine_body(d1_hbm, d2_hbm, ys_hbm, out_hbm,
                  i1_all, i2_all, y1a, y1b, y2a, y2b,
                  sg1a, sg1b, sg2a, sg2b, ssa, ssb):
    wid = lax.axis_index("s") * NC + lax.axis_index("c")
    T = d1_hbm.shape[0]
    tpw = T // NW
    nch = tpw // CCH
    H = y1a.shape[1]
    base = wid * tpw
    pltpu.sync_copy(d1_hbm.at[pl.ds(base, tpw)], i1_all)
    pltpu.sync_copy(d2_hbm.at[pl.ds(base, tpw)], i2_all)
    y1 = (y1a, y1b)
    y2 = (y2a, y2b)
    sg1 = (sg1a, sg1b)
    sg2 = (sg2a, sg2b)
    ss = (ssa, ssb)

    def g(c):
        p = c % 2
        return (pltpu.async_copy(ys_hbm.at[i1_all.at[pl.ds(c * CCH, CCH)]],
                                 y1[p], sg1[p]),
                pltpu.async_copy(ys_hbm.at[i2_all.at[pl.ds(c * CCH, CCH)]],
                                 y2[p], sg2[p]))

    gd = [None] * nch
    sd = [None] * nch
    gd[0] = g(0)
    for c in range(nch):
        p = c % 2
        gd[c][0].wait()
        gd[c][1].wait()
        if c + 1 < nch:
            if c >= 1:
                sd[c - 1].wait()
            gd[c + 1] = g(c + 1)
        for r in range(CCH):

            def inner(jv, carry, r=r, p=p):
                for u in range(UNR):
                    sl = pl.ds((jv * UNR + u) * LANES, LANES)
                    y1[p][r, sl] = y1[p][r, sl] + y2[p][r, sl]
                return carry

            lax.fori_loop(0, H // (LANES * UNR), inner, 0)
        sd[c] = pltpu.async_copy(
            y1[p], out_hbm.at[pl.ds(base + c * CCH, CCH)], ss[p])
    sd[nch - 2].wait()
    sd[nch - 1].wait()


def _gffn_body(be_ref, nbt_ref, tid_ref, gp_ref, up_ref, dp_ref, ws_ref,
               x_hbm, ys_ref, xbuf, sem0, sem1):
    i = pl.program_id(0)
    BTc, H = ys_ref.shape

    def issue(blk, par, sem):
        def one(r8, carry):
            for u in range(8):
                r = r8 * 8 + u
                t = tid_ref[blk * BTc + r]
                pltpu.make_async_copy(
                    x_hbm.at[pl.ds(t, 1)],
                    xbuf.at[pl.ds(par * BTc + r, 1)], sem).start()
            return carry
        lax.fori_loop(0, BTc // 8, one, 0)

    def drain(par, sem):
        pltpu.make_async_copy(
            x_hbm.at[pl.ds(0, BTc)],
            xbuf.at[pl.ds(par * BTc, BTc)], sem).wait()

    nbt = nbt_ref[0]

    @pl.when(i == 0)
    def _():
        issue(0, 0, sem0)

    @pl.when((i + 1 < nbt) & (i % 2 == 0))
    def _():
        issue(i + 1, 1, sem1)

    @pl.when((i + 1 < nbt) & (i % 2 == 1))
    def _():
        issue(i + 1, 0, sem0)

    @pl.when(i < nbt)
    def _():

        @pl.when(i % 2 == 0)
        def _():
            drain(0, sem0)

        @pl.when(i % 2 == 1)
        def _():
            drain(1, sem1)

        x = xbuf[pl.ds((i % 2) * BTc, BTc), :]
        g = jax.lax.dot_general(x, gp_ref[0], (((1,), (1,)), ((), ())),
                                preferred_element_type=jnp.float32)
        u = jax.lax.dot_general(x, up_ref[0], (((1,), (1,)), ((), ())),
                                preferred_element_type=jnp.float32)
        h = g * jax.nn.sigmoid(g) * u
        y = jax.lax.dot_general(h, dp_ref[0], (((1,), (1,)), ((), ())),
                                preferred_element_type=jnp.float32)
        ys_ref[...] = y * ws_ref[...][:, 0:1]


@functools.partial(jax.jit, static_argnames=("interpret",))
def kernel(hidden_states, gate_w, gate_proj, up_proj, down_proj,
           expert_bias, interpret=False):
    B, S, H = hidden_states.shape
    T = B * S
    FF = gate_proj.shape[1]
    NPAD = NBMAX * BT
    x = hidden_states.reshape(T, H)

    gw_pad = jnp.zeros((EP, H), jnp.float32).at[:E].set(gate_w)
    bias_pad = jnp.full((1, EP), NEG, jnp.float32).at[0, :E].set(expert_bias)

    iw, ww = pl.pallas_call(
        _router_body,
        out_shape=(jax.ShapeDtypeStruct((T, EP), jnp.int32),
                   jax.ShapeDtypeStruct((T, EP), jnp.float32)),
        interpret=interpret,
    )(x, gw_pad, bias_pad)

    tid, wsl, be, nbt, dst1, dst2 = _route_meta(iw, ww, T)

    wsb = jnp.broadcast_to(wsl[:, None], (NPAD, EP))
    grid_spec = pltpu.PrefetchScalarGridSpec(
        num_scalar_prefetch=3,
        grid=(NBMAX,),
        in_specs=[
            pl.BlockSpec((1, FF, H), lambda i, be, nbt, tid: (be[i], 0, 0)),
            pl.BlockSpec((1, FF, H), lambda i, be, nbt, tid: (be[i], 0, 0)),
            pl.BlockSpec((1, H, FF), lambda i, be, nbt, tid: (be[i], 0, 0)),
            pl.BlockSpec((BT, EP), lambda i, be, nbt, tid: (i, 0)),
            pl.BlockSpec(memory_space=pl.ANY),
        ],
        out_specs=pl.BlockSpec((BT, H), lambda i, be, nbt, tid: (i, 0)),
        scratch_shapes=[
            pltpu.VMEM((2 * BT, H), jnp.float32),
            pltpu.SemaphoreType.DMA,
            pltpu.SemaphoreType.DMA,
        ],
    )
    ys = pl.pallas_call(
        _gffn_body,
        grid_spec=grid_spec,
        out_shape=jax.ShapeDtypeStruct((NPAD, H), jnp.float32),
        compiler_params=pltpu.CompilerParams(
            dimension_semantics=("arbitrary",)),
        interpret=interpret,
    )(be, nbt, tid, gate_proj, up_proj, down_proj, wsb, x)

    combine = pl.kernel(
        _combine_body,
        out_type=jax.ShapeDtypeStruct((T, H), jnp.float32),
        mesh=plsc.VectorSubcoreMesh(core_axis_name="c", subcore_axis_name="s"),
        scratch_types=[
            pltpu.VMEM((T // NW,), jnp.int32),
            pltpu.VMEM((T // NW,), jnp.int32),
            pltpu.VMEM((CCH, H), jnp.float32),
            pltpu.VMEM((CCH, H), jnp.float32),
            pltpu.VMEM((CCH, H), jnp.float32),
            pltpu.VMEM((CCH, H), jnp.float32),
            pltpu.SemaphoreType.DMA,
            pltpu.SemaphoreType.DMA,
            pltpu.SemaphoreType.DMA,
            pltpu.SemaphoreType.DMA,
            pltpu.SemaphoreType.DMA,
            pltpu.SemaphoreType.DMA,
        ],
    )
    out = combine(dst1, dst2, ys)

    return out.reshape(B, S, H)


# speedup vs baseline: 1.4334x; 1.0013x over previous
import jax
import jax.numpy as jnp
from jax import lax
from jax.experimental import pallas as pl
from jax.experimental.pallas import tpu as pltpu
from jax.experimental.pallas import tpu_sc as plsc

E = 8
EP = 128
NEG = -1e30

BT = 256
NBMAX = 2 * 2048 // BT + E

NC, NS, LANES = 2, 16, 16
NW = NC * NS


def _router_body(x_ref, gw_ref, bias_ref, iw_ref, ww_ref):
    x = x_ref[...]
    logits = jax.lax.dot_general(
        x, gw_ref[...], (((1,), (1,)), ((), ())),
        preferred_element_type=jnp.float32)
    rw = jax.nn.sigmoid(logits)
    scores = rw + bias_ref[...]
    idx = jax.lax.broadcasted_iota(jnp.int32, scores.shape, 1)
    m1 = jnp.max(scores, axis=1, keepdims=True)
    i1 = jnp.min(jnp.where(scores == m1, idx, EP), axis=1, keepdims=True)
    sel1 = idx == i1
    s2 = jnp.where(sel1, NEG, scores)
    m2 = jnp.max(s2, axis=1, keepdims=True)
    i2 = jnp.min(jnp.where(s2 == m2, idx, EP), axis=1, keepdims=True)
    sel2 = idx == i2
    w1 = jnp.sum(jnp.where(sel1, rw, 0.0), axis=1, keepdims=True)
    w2 = jnp.sum(jnp.where(sel2, rw, 0.0), axis=1, keepdims=True)
    den = w1 + w2 + 1e-6
    lane0 = idx == 0
    lane1 = idx == 1
    iw_ref[...] = (jnp.where(lane0, i1, 0) + jnp.where(lane1, i2, 0))
    ww_ref[...] = (jnp.where(lane0, w1 / den, 0.0)
                   + jnp.where(lane1, w2 / den, 0.0))


def _route_meta(iw, ww, T):
    e1, e2 = iw[:, 0], iw[:, 1]
    npad = NBMAX * BT
    ar = jnp.arange(E, dtype=jnp.int32)
    oh1 = (e1[:, None] == ar[None, :]).astype(jnp.int32)
    oh2 = (e2[:, None] == ar[None, :]).astype(jnp.int32)
    c1 = jnp.cumsum(oh1, axis=0)
    c2 = jnp.cumsum(oh2, axis=0)
    cnt1, cnt2 = c1[-1], c2[-1]
    rank1 = jnp.take_along_axis(c1, e1[:, None], 1)[:, 0] - 1
    rank2 = cnt1[e2] + jnp.take_along_axis(c2, e2[:, None], 1)[:, 0] - 1
    count = cnt1 + cnt2
    nb = (count + BT - 1) // BT
    bstart = jnp.concatenate(
        [jnp.zeros(1, jnp.int32), jnp.cumsum(nb)[:-1].astype(jnp.int32)])
    nbt = jnp.sum(nb).astype(jnp.int32)[None]
    pad_start = BT * bstart
    dst1 = pad_start[e1] + rank1
    dst2 = pad_start[e2] + rank2
    tok = jnp.arange(T, dtype=jnp.int32)
    tid = (jnp.zeros(npad, jnp.int32).at[dst1].set(tok).at[dst2].set(tok))
    wsl = (jnp.zeros(npad, jnp.float32).at[dst1].set(ww[:, 0])
           .at[dst2].set(ww[:, 1]))
    j = jnp.arange(NBMAX, dtype=jnp.int32)
    be = jnp.sum((j[:, None] >= bstart[None, :]).astype(jnp.int32), 1) - 1
    return (tid, wsl, be, nbt,
            dst1.astype(jnp.int32), dst2.astype(jnp.int32))


CCH = 8
UNR = 4


def _combine_body(d1_hbm, d2_hbm, ys_hbm, out_hbm,
                  i1_all, i2_all, y1a, y1b, y2a, y2b,
                  sg1a, sg1b, sg2a, sg2b, ssa, ssb):
    wid = lax.axis_index("s") * NC + lax.axis_index("c")
    T = d1_hbm.shape[0]
    tpw = T // NW
    nch = tpw // CCH
    H = y1a.shape[1]
    base = wid * tpw
    pltpu.sync_copy(d1_hbm.at[pl.ds(base, tpw)], i1_all)
    pltpu.sync_copy(d2_hbm.at[pl.ds(base, tpw)], i2_all)
    y1 = (y1a, y1b)
    y2 = (y2a, y2b)
    sg1 = (sg1a, sg1b)
    sg2 = (sg2a, sg2b)
    ss = (ssa, ssb)

    def g(c):
        p = c % 2
        return (pltpu.async_copy(ys_hbm.at[i1_all.at[pl.ds(c * CCH, CCH)]],
                                 y1[p], sg1[p]),
                pltpu.async_copy(ys_hbm.at[i2_all.at[pl.ds(c * CCH, CCH)]],
                                 y2[p], sg2[p]))

    gd = [None] * nch
    sd = [None] * nch
    gd[0] = g(0)
    for c in range(nch):
        p = c % 2
        gd[c][0].wait()
        gd[c][1].wait()
        if c + 1 < nch:
            if c >= 1:
                sd[c - 1].wait()
            gd[c + 1] = g(c + 1)
        for r in range(CCH):

            def inner(jv, carry, r=r, p=p):
                for u in range(UNR):
                    sl = pl.ds((jv * UNR + u) * LANES, LANES)
                    y1[p][r, sl] = y1[p][r, sl] + y2[p][r, sl]
                return carry

            lax.fori_loop(0, H // (LANES * UNR), inner, 0)
        sd[c] = pltpu.async_copy(
            y1[p], out_hbm.at[pl.ds(base + c * CCH, CCH)], ss[p])
    sd[nch - 2].wait()
    sd[nch - 1].wait()


def _gffn_body(be_ref, nbt_ref, tid_ref, gp_ref, up_ref, dp_ref, ws_ref,
               x_hbm, ys_ref, xbuf, sem0, sem1):
    i = pl.program_id(0)
    BTc, H = ys_ref.shape

    def issue(blk, par, sem):
        def one(r8, carry):
            for u in range(8):
                r = r8 * 8 + u
                t = tid_ref[blk * BTc + r]
                pltpu.make_async_copy(
                    x_hbm.at[pl.ds(t, 1)],
                    xbuf.at[pl.ds(par * BTc + r, 1)], sem).start()
            return carry
        lax.fori_loop(0, BTc // 8, one, 0)

    def drain(par, sem):
        pltpu.make_async_copy(
            x_hbm.at[pl.ds(0, BTc)],
            xbuf.at[pl.ds(par * BTc, BTc)], sem).wait()

    nbt = nbt_ref[0]

    @pl.when(i == 0)
    def _():
        issue(0, 0, sem0)

    @pl.when((i + 1 < nbt) & (i % 2 == 0))
    def _():
        issue(i + 1, 1, sem1)

    @pl.when((i + 1 < nbt) & (i % 2 == 1))
    def _():
        issue(i + 1, 0, sem0)

    @pl.when(i < nbt)
    def _():

        @pl.when(i % 2 == 0)
        def _():
            drain(0, sem0)

        @pl.when(i % 2 == 1)
        def _():
            drain(1, sem1)

        x = xbuf[pl.ds((i % 2) * BTc, BTc), :]
        g = jax.lax.dot_general(x, gp_ref[0], (((1,), (1,)), ((), ())),
                                preferred_element_type=jnp.float32)
        u = jax.lax.dot_general(x, up_ref[0], (((1,), (1,)), ((), ())),
                                preferred_element_type=jnp.float32)
        h = g * jax.nn.sigmoid(g) * u
        y = jax.lax.dot_general(h, dp_ref[0], (((1,), (1,)), ((), ())),
                                preferred_element_type=jnp.float32)
        ys_ref[...] = y * ws_ref[...][:, 0:1]


@jax.jit
def kernel(hidden_states, gate_w, gate_proj, up_proj, down_proj,
           expert_bias):
    B, S, H = hidden_states.shape
    T = B * S
    FF = gate_proj.shape[1]
    NPAD = NBMAX * BT
    x = hidden_states.reshape(T, H)

    gw_pad = jnp.zeros((EP, H), jnp.float32).at[:E].set(gate_w)
    bias_pad = jnp.full((1, EP), NEG, jnp.float32).at[0, :E].set(expert_bias)

    iw, ww = pl.pallas_call(
        _router_body,
        out_shape=(jax.ShapeDtypeStruct((T, EP), jnp.int32),
                   jax.ShapeDtypeStruct((T, EP), jnp.float32)),
    )(x, gw_pad, bias_pad)

    tid, wsl, be, nbt, dst1, dst2 = _route_meta(iw, ww, T)

    wsb = jnp.broadcast_to(wsl[:, None], (NPAD, EP))
    grid_spec = pltpu.PrefetchScalarGridSpec(
        num_scalar_prefetch=3,
        grid=(NBMAX,),
        in_specs=[
            pl.BlockSpec((1, FF, H), lambda i, be, nbt, tid: (be[i], 0, 0)),
            pl.BlockSpec((1, FF, H), lambda i, be, nbt, tid: (be[i], 0, 0)),
            pl.BlockSpec((1, H, FF), lambda i, be, nbt, tid: (be[i], 0, 0)),
            pl.BlockSpec((BT, EP), lambda i, be, nbt, tid: (i, 0)),
            pl.BlockSpec(memory_space=pl.ANY),
        ],
        out_specs=pl.BlockSpec((BT, H), lambda i, be, nbt, tid: (i, 0)),
        scratch_shapes=[
            pltpu.VMEM((2 * BT, H), jnp.float32),
            pltpu.SemaphoreType.DMA,
            pltpu.SemaphoreType.DMA,
        ],
    )
    ys = pl.pallas_call(
        _gffn_body,
        grid_spec=grid_spec,
        out_shape=jax.ShapeDtypeStruct((NPAD, H), jnp.float32),
        compiler_params=pltpu.CompilerParams(
            dimension_semantics=("arbitrary",)),
    )(be, nbt, tid, gate_proj, up_proj, down_proj, wsb, x)

    combine = pl.kernel(
        _combine_body,
        out_type=jax.ShapeDtypeStruct((T, H), jnp.float32),
        mesh=plsc.VectorSubcoreMesh(core_axis_name="c", subcore_axis_name="s"),
        scratch_types=[
            pltpu.VMEM((T // NW,), jnp.int32),
            pltpu.VMEM((T // NW,), jnp.int32),
            pltpu.VMEM((CCH, H), jnp.float32),
            pltpu.VMEM((CCH, H), jnp.float32),
            pltpu.VMEM((CCH, H), jnp.float32),
            pltpu.VMEM((CCH, H), jnp.float32),
            pltpu.SemaphoreType.DMA,
            pltpu.SemaphoreType.DMA,
            pltpu.SemaphoreType.DMA,
            pltpu.SemaphoreType.DMA,
            pltpu.SemaphoreType.DMA,
            pltpu.SemaphoreType.DMA,
        ],
    )
    out = combine(dst1, dst2, ys)

    return out.reshape(B, S, H)
